# baseline (speedup 1.0000x reference)
import jax
import jax.numpy as jnp
from jax.experimental import pallas as pl
from jax.experimental.pallas import tpu as pltpu

DICT_LEN = 9
NUM_LS = 8
DIM = 1024
SEQ = 2048
BZ = 16
BLK_S = 512
BLK_B = 4


def _mapper_kernel(idx_ref, x_ref, w_ref, b_ref, o_ref, xcol, sem):
    bh = pl.program_id(0)

    def copy(j, slot):
        return pltpu.make_async_copy(
            x_ref.at[:, 0, j, :], xcol.at[slot], sem.at[slot]
        )

    copy(0, 0).start()
    for j in range(BLK_B):
        slot = j % 2
        if j + 1 < BLK_B:
            copy(j + 1, 1 - slot).start()
        copy(j, slot).wait()
        e = idx_ref[bh * BLK_B + j]
        xj = xcol[slot].astype(jnp.bfloat16)
        yj = jax.lax.dot_general(
            xj, w_ref[e],
            dimension_numbers=(((1,), (0,)), ((), ())),
            preferred_element_type=jnp.float32,
        )
        o_ref[:, 0, j, :] = yj + b_ref[e]


def kernel(x, lang_ids, W, b):
    idx = jnp.clip(DICT_LEN - 2 - lang_ids, 0, NUM_LS - 1).astype(jnp.int32)
    Wt = jnp.swapaxes(W, 1, 2).astype(jnp.bfloat16)
    x4 = x.reshape(SEQ, BZ // BLK_B, BLK_B, DIM)
    grid = (BZ // BLK_B, SEQ // BLK_S)
    out = pl.pallas_call(
        _mapper_kernel,
        grid_spec=pltpu.PrefetchScalarGridSpec(
            num_scalar_prefetch=1,
            grid=grid,
            in_specs=[
                pl.BlockSpec((BLK_S, 1, BLK_B, DIM), lambda bh, s, idx_ref: (s, bh, 0, 0)),
                pl.BlockSpec((NUM_LS, DIM, DIM), lambda bh, s, idx_ref: (0, 0, 0)),
                pl.BlockSpec((NUM_LS, DIM), lambda bh, s, idx_ref: (0, 0)),
            ],
            out_specs=pl.BlockSpec((BLK_S, 1, BLK_B, DIM), lambda bh, s, idx_ref: (s, bh, 0, 0)),
            scratch_shapes=[
                pltpu.VMEM((2, BLK_S, DIM), jnp.float32),
                pltpu.SemaphoreType.DMA((2,)),
            ],
        ),
        out_shape=jax.ShapeDtypeStruct((SEQ, BZ // BLK_B, BLK_B, DIM), jnp.float32),
    )(idx, x4, Wt, b)
    return out.reshape(SEQ, BZ, DIM)
